# baseline (speedup 1.0000x reference)
import jax
import jax.numpy as jnp
from jax.experimental import pallas as pl
from jax.experimental.pallas import tpu as pltpu

N, H, W, C = 4, 224, 224, 96
TB = 16


def _upsample_body(prev_ref, mid_ref, next_ref, out_ref):
    for r in range(TB):
        prow = mid_ref[0, r - 1] if r >= 1 else prev_ref[0, 0]
        crow = mid_ref[0, r]
        nrow = mid_ref[0, r + 1] if r < TB - 1 else next_ref[0, 0]
        for a, bl in ((0, 0.25 * prow + 0.75 * crow),
                      (1, 0.75 * crow + 0.25 * nrow)):
            sp = jnp.concatenate([bl[:1], bl[:-1]], axis=0)
            sn = jnp.concatenate([bl[1:], bl[-1:]], axis=0)
            e = 0.25 * sp + 0.75 * bl
            o = 0.75 * bl + 0.25 * sn
            out_ref[0, r, a] = jnp.concatenate([e, o], axis=1)


def kernel(img):
    nblk = H // TB
    out5 = pl.pallas_call(
        _upsample_body,
        grid=(N, nblk),
        in_specs=[
            pl.BlockSpec((1, 1, W, C),
                         lambda n, t: (n, jnp.maximum(t * TB - 1, 0), 0, 0)),
            pl.BlockSpec((1, TB, W, C), lambda n, t: (n, t, 0, 0)),
            pl.BlockSpec((1, 1, W, C),
                         lambda n, t: (n, jnp.minimum(t * TB + TB, H - 1), 0, 0)),
        ],
        out_specs=pl.BlockSpec((1, TB, 2, W, 2 * C),
                               lambda n, t: (n, t, 0, 0, 0)),
        out_shape=jax.ShapeDtypeStruct((N, H, 2, W, 2 * C), img.dtype),
        compiler_params=pltpu.CompilerParams(
            dimension_semantics=("parallel", "arbitrary")),
    )(img, img, img)
    return out5.reshape(N, 2 * H, 2 * W, C)
